# baseline (speedup 1.0000x reference)
import functools

import jax
import jax.numpy as jnp
from jax import lax
from jax.experimental import pallas as pl
from jax.experimental.pallas import tpu as pltpu
from jax.experimental.pallas import tpu_sc as plsc

N = 10000
H = 64
G = 128
NC = 2
NS = 16
NW = NC * NS
CH = 80
NPAD = 10240
RPT = NPAD // NS
NCH_N = N // CH
NB = 40960


def _sc_mesh():
    return plsc.VectorSubcoreMesh(core_axis_name="c", subcore_axis_name="s")


SW = 8


@functools.lru_cache(maxsize=None)
def _stats_kernel(E):
    Et = E // NW
    nch = Et // CH

    assert nch % NBUF == 0 and nch // NBUF >= 2

    @functools.partial(
        pl.kernel,
        out_type=(jax.ShapeDtypeStruct((NC, NPAD, SW), jnp.float32),
                  jax.ShapeDtypeStruct((NC, G, SW), jnp.float32)),
        mesh=_sc_mesh(),
        compiler_params=pltpu.CompilerParams(use_tc_tiling_on_sc=False),
        scratch_types=[
            pltpu.VMEM_SHARED((NPAD, SW), jnp.float32),
            pltpu.VMEM_SHARED((G, SW), jnp.float32),
            pltpu.VMEM((nch, CH), jnp.int32),
            pltpu.VMEM((CH,), jnp.int32),
            pltpu.VMEM((CH, SW), jnp.float32),
            pltpu.SemaphoreType.DMA((NBUF,)),
        ],
    )
    def stats_kernel(dst_hbm, seg_hbm, zeros_hbm, ones_hbm, deg_out,
                     cnt_out, deg_s, cnt_s, idx2, idx, ones_b, ssem):
        c = lax.axis_index("c")
        s = lax.axis_index("s")
        w = s * NC + c

        pltpu.sync_copy(ones_hbm, ones_b)
        pltpu.sync_copy(zeros_hbm, deg_s.at[pl.ds(s * RPT, RPT)])

        @pl.when(s == 0)
        def _():
            pltpu.sync_copy(zeros_hbm.at[pl.ds(0, G)], cnt_s)

        rowbase = (c * (E // NC) + s * Et) // CH
        pltpu.sync_copy(dst_hbm.at[pl.ds(rowbase, nch)], idx2)
        plsc.subcore_barrier()

        def start_scatter(j, u):
            pltpu.async_copy(ones_b, deg_s.at[idx2.at[j]], ssem.at[u],
                             add=True)

        def wait_scatter(j, u):
            pltpu.make_async_copy(ones_b, deg_s.at[idx2.at[j]],
                                  ssem.at[u]).wait()

        for u in range(NBUF):
            start_scatter(u, u)

        def steady(t, _):
            for u in range(NBUF):
                j = t * NBUF + u
                wait_scatter(j - NBUF, u)
                start_scatter(j, u)
            return 0

        lax.fori_loop(1, nch // NBUF, steady, 0)
        for u in range(NBUF):
            wait_scatter(nch - NBUF + u, u)

        def cnt_body(j, _):
            base = (w + j * NW) * CH
            pltpu.sync_copy(seg_hbm.at[pl.ds(base, CH)], idx)
            pltpu.sync_copy(ones_b, cnt_s.at[idx], add=True)
            return 0

        lax.fori_loop(0, (NCH_N - w + NW - 1) // NW, cnt_body, 0)
        plsc.subcore_barrier()
        pltpu.sync_copy(deg_s.at[pl.ds(s * RPT, RPT)],
                        deg_out.at[c, pl.ds(s * RPT, RPT)])

        @pl.when(s == 0)
        def _():
            pltpu.sync_copy(cnt_s, cnt_out.at[c])

    return stats_kernel


NBUF = 5


@functools.lru_cache(maxsize=None)
def _agg_kernel(E):
    Et = E // NW
    nch = Et // CH
    assert nch % NBUF == 0 and nch // NBUF >= 2

    @functools.partial(
        pl.kernel,
        out_type=jax.ShapeDtypeStruct((NC, NPAD, H), jnp.float32),
        mesh=_sc_mesh(),
        compiler_params=pltpu.CompilerParams(use_tc_tiling_on_sc=False),
        scratch_types=[
            pltpu.VMEM_SHARED((NPAD, H), jnp.float32),
            pltpu.VMEM((nch, CH), jnp.int32),
            pltpu.VMEM((nch, CH), jnp.int32),
            pltpu.VMEM((NBUF, CH, H), jnp.float32),
            pltpu.VMEM((RPT, H), jnp.float32),
            pltpu.SemaphoreType.DMA((NBUF,)),
            pltpu.SemaphoreType.DMA((NBUF,)),
        ],
    )
    def agg_kernel(y_hbm, src_hbm, dst_hbm, out_hbm, acc_s,
                   idx_s, idx_d, rows, zbuf, gsem, ssem):
        c = lax.axis_index("c")
        s = lax.axis_index("s")

        def fill_z(i, _):
            for k in range(H // 16):
                zbuf[i, pl.ds(k * 16, 16)] = jnp.zeros((16,), jnp.float32)
            return 0

        lax.fori_loop(0, RPT, fill_z, 0)
        pltpu.sync_copy(zbuf, acc_s.at[pl.ds(s * RPT, RPT)])

        rowbase = (c * (E // NC) + s * Et) // CH
        pltpu.sync_copy(src_hbm.at[pl.ds(rowbase, nch)], idx_s)
        pltpu.sync_copy(dst_hbm.at[pl.ds(rowbase, nch)], idx_d)
        plsc.subcore_barrier()

        def start_gather(j, u):
            pltpu.async_copy(y_hbm.at[idx_s.at[j]], rows.at[u], gsem.at[u])

        def wait_gather(j, u):
            pltpu.make_async_copy(y_hbm.at[idx_s.at[j]], rows.at[u],
                                  gsem.at[u]).wait()

        def start_scatter(j, u):
            pltpu.async_copy(rows.at[u], acc_s.at[idx_d.at[j]], ssem.at[u],
                             add=True)

        def wait_scatter(j, u):
            pltpu.make_async_copy(rows.at[u], acc_s.at[idx_d.at[j]],
                                  ssem.at[u]).wait()

        LA = 4
        for u in range(LA):
            start_gather(u, u)
        for u in range(NBUF):
            j = u
            wait_gather(j, u)
            start_scatter(j, u)
            if j + LA < nch:
                u2 = (u + LA) % NBUF
                if j + LA >= NBUF:
                    wait_scatter(j + LA - NBUF, u2)
                start_gather(j + LA, u2)

        def steady(t, _):
            for u in range(NBUF):
                j = t * NBUF + u
                wait_gather(j, u)
                start_scatter(j, u)
                u2 = (u + LA) % NBUF
                wait_scatter(j + LA - NBUF, u2)
                start_gather(j + LA, u2)
            return 0

        lax.fori_loop(1, nch // NBUF - 1, steady, 0)

        for u in range(NBUF):
            j = nch - NBUF + u
            wait_gather(j, u)
            start_scatter(j, u)
            if j + LA < nch:
                u2 = (u + LA) % NBUF
                wait_scatter(j + LA - NBUF, u2)
                start_gather(j + LA, u2)
        for u in range(NBUF):
            wait_scatter(nch - NBUF + u, u)

        plsc.subcore_barrier()
        pltpu.sync_copy(acc_s.at[pl.ds(s * RPT, RPT)],
                        out_hbm.at[c, pl.ds(s * RPT, RPT)])

    return agg_kernel


@functools.lru_cache(maxsize=None)
def _agg_pool_kernel(E):
    Et = E // NW
    nch = Et // CH
    assert nch % NBUF == 0 and nch // NBUF >= 2

    @functools.partial(
        pl.kernel,
        out_type=jax.ShapeDtypeStruct((NC, G, H), jnp.float32),
        mesh=_sc_mesh(),
        compiler_params=pltpu.CompilerParams(use_tc_tiling_on_sc=False),
        scratch_types=[
            pltpu.VMEM_SHARED((NPAD, H), jnp.float32),
            pltpu.VMEM_SHARED((G, H), jnp.float32),
            pltpu.VMEM((nch, CH), jnp.int32),
            pltpu.VMEM((nch, CH), jnp.int32),
            pltpu.VMEM((NBUF, CH, H), jnp.float32),
            pltpu.VMEM((2, CH, 16), jnp.float32),
            pltpu.VMEM((2, CH), jnp.int32),
            pltpu.VMEM((RPT, H), jnp.float32),
            pltpu.SemaphoreType.DMA((NBUF,)),
            pltpu.SemaphoreType.DMA((NBUF,)),
        ],
    )
    def agg_pool_kernel(y_hbm, src_hbm, dst_hbm, dv_hbm, seg_hbm,
                        out_hbm, acc_s, pool_s, idx_s, idx_d,
                        rows, dbuf, sidx, zbuf, gsem, ssem):
        c = lax.axis_index("c")
        s = lax.axis_index("s")

        def fill_z(i, _):
            for k in range(H // 16):
                zbuf[i, pl.ds(k * 16, 16)] = jnp.zeros((16,), jnp.float32)
            return 0

        lax.fori_loop(0, RPT, fill_z, 0)
        pltpu.sync_copy(zbuf, acc_s.at[pl.ds(s * RPT, RPT)])

        @pl.when(s == 0)
        def _():
            pltpu.sync_copy(zbuf.at[pl.ds(0, G)], pool_s)

        rowbase = (c * (E // NC) + s * Et) // CH
        pltpu.sync_copy(src_hbm.at[pl.ds(rowbase, nch)], idx_s)
        pltpu.sync_copy(dst_hbm.at[pl.ds(rowbase, nch)], idx_d)
        plsc.subcore_barrier()

        def start_gather(j, u):
            pltpu.async_copy(y_hbm.at[idx_s.at[j]], rows.at[u], gsem.at[u])

        def wait_gather(j, u):
            pltpu.make_async_copy(y_hbm.at[idx_s.at[j]], rows.at[u],
                                  gsem.at[u]).wait()

        def start_scatter(j, u):
            pltpu.async_copy(rows.at[u], acc_s.at[idx_d.at[j]], ssem.at[u],
                             add=True)

        def wait_scatter(j, u):
            pltpu.make_async_copy(rows.at[u], acc_s.at[idx_d.at[j]],
                                  ssem.at[u]).wait()

        LA = 4
        for u in range(LA):
            start_gather(u, u)
        for u in range(NBUF):
            j = u
            wait_gather(j, u)
            start_scatter(j, u)
            if j + LA < nch:
                u2 = (u + LA) % NBUF
                if j + LA >= NBUF:
                    wait_scatter(j + LA - NBUF, u2)
                start_gather(j + LA, u2)

        def steady(t, _):
            for u in range(NBUF):
                j = t * NBUF + u
                wait_gather(j, u)
                start_scatter(j, u)
                u2 = (u + LA) % NBUF
                wait_scatter(j + LA - NBUF, u2)
                start_gather(j + LA, u2)
            return 0

        lax.fori_loop(1, nch // NBUF - 1, steady, 0)

        for u in range(NBUF):
            j = nch - NBUF + u
            wait_gather(j, u)
            start_scatter(j, u)
            if j + LA < nch:
                u2 = (u + LA) % NBUF
                wait_scatter(j + LA - NBUF, u2)
                start_gather(j + LA, u2)
        for u in range(NBUF):
            wait_scatter(nch - NBUF + u, u)

        plsc.subcore_barrier()

        nk = (NCH_N - s + NS - 1) // NS
        a = rows.at[0]
        yb = rows.at[1]
        d = dbuf.at[0]
        si = sidx.at[0]

        def pool_body(i, _):
            k = s + i * NS
            base = k * CH
            pltpu.sync_copy(acc_s.at[pl.ds(base, CH)], a)
            pltpu.sync_copy(dv_hbm.at[pl.ds(base, CH)], d)
            pltpu.sync_copy(seg_hbm.at[k], si)

            @pl.when(c == 0)
            def _():
                pltpu.sync_copy(y_hbm.at[pl.ds(base, CH)], yb)

                def ew(r2, _):
                    for rr in range(2):
                        r = r2 * 2 + rr
                        dv = d[r, :]
                        for k4 in range(H // 16):
                            sl = pl.ds(k4 * 16, 16)
                            a[r, sl] = (a[r, sl] + yb[r, sl]) * dv
                    return 0

                lax.fori_loop(0, CH // 2, ew, 0)

            @pl.when(c != 0)
            def _():
                def ew(r2, _):
                    for rr in range(2):
                        r = r2 * 2 + rr
                        dv = d[r, :]
                        for k4 in range(H // 16):
                            sl = pl.ds(k4 * 16, 16)
                            a[r, sl] = a[r, sl] * dv
                    return 0

                lax.fori_loop(0, CH // 2, ew, 0)

            pltpu.sync_copy(a, pool_s.at[si], add=True)
            return 0

        lax.fori_loop(0, nk, pool_body, 0)
        plsc.subcore_barrier()

        @pl.when(s == 0)
        def _():
            pltpu.sync_copy(pool_s, out_hbm.at[c])

    return agg_pool_kernel


def _tc_scale_matmul(x, W1, degp):
    def body(x_ref, w_ref, degp_ref, y_ref, dinv_ref):
        deg = degp_ref[0, 0:N, 0:1] + degp_ref[1, 0:N, 0:1] + 1.0
        dinv = lax.rsqrt(deg)
        y_ref[0:N, :] = jnp.dot(x_ref[...], w_ref[...],
                                preferred_element_type=jnp.float32) * dinv
        dinv_ref[...] = jnp.broadcast_to(dinv, (N, 16))

    return pl.pallas_call(
        body,
        out_shape=[jax.ShapeDtypeStruct((NB, H), jnp.float32),
                   jax.ShapeDtypeStruct((N, 16), jnp.float32)],
    )(x, W1, degp)


def _tc_mid(p, y1, dinv8, b1, W2):
    def body(p_ref, y_ref, dinv_ref, b_ref, w_ref, y2_ref):
        dinv = dinv_ref[:, 0:1]
        h = dinv * (p_ref[0, 0:N, :] + p_ref[1, 0:N, :] + y_ref[0:N, :]) + b_ref[...]
        h = jnp.maximum(h, 0.0)
        y2_ref[0:N, :] = jnp.dot(h, w_ref[...],
                                 preferred_element_type=jnp.float32) * dinv

    return pl.pallas_call(
        body,
        out_shape=jax.ShapeDtypeStruct((NB, H), jnp.float32),
    )(p, y1, dinv8, b1, W2)


def _tc_head(poolp, cntp, b2, Wl, bl):
    def body(pp_ref, cnt_ref, b_ref, wl_ref, bl_ref, o_ref):
        cnt = cnt_ref[0, :, 0:1] + cnt_ref[1, :, 0:1]
        sums = pp_ref[0] + pp_ref[1] + cnt * b_ref[...]
        pooled = sums / jnp.maximum(cnt, 1.0)
        o_ref[...] = jnp.dot(pooled, wl_ref[...],
                             preferred_element_type=jnp.float32) + bl_ref[...]

    return pl.pallas_call(
        body,
        out_shape=jax.ShapeDtypeStruct((G, 1), jnp.float32),
    )(poolp, cntp, b2, Wl, bl)


def kernel(x, edge_index, batch, num_graphs, W1, b1, W2, b2, Wl, bl):
    src = edge_index[0]
    dst = edge_index[1]
    E = src.shape[0]
    seg = jnp.minimum(batch, num_graphs - 1).astype(jnp.int32)

    src2 = src.reshape(E // CH, CH)
    dst2 = dst.reshape(E // CH, CH)
    zeros8 = jnp.zeros((RPT, SW), jnp.float32)
    ones8 = jnp.ones((CH, SW), jnp.float32)
    degp, cntp = _stats_kernel(E)(dst2, seg, zeros8, ones8)
    y1, dinvb = _tc_scale_matmul(x, W1, degp)
    p1 = _agg_kernel(E)(y1, src2, dst2)
    y2 = _tc_mid(p1, y1, dinvb, b1.reshape(1, H), W2)
    seg2 = seg.reshape(NCH_N, CH)
    poolp = _agg_pool_kernel(E)(y2, src2, dst2, dinvb, seg2)
    return _tc_head(poolp, cntp, b2.reshape(1, H), Wl, bl.reshape(1, 1))


# speedup vs baseline: 1.1702x; 1.1702x over previous
import functools

import jax
import jax.numpy as jnp
from jax import lax
from jax.experimental import pallas as pl
from jax.experimental.pallas import tpu as pltpu
from jax.experimental.pallas import tpu_sc as plsc

N = 10000
H = 64
G = 128
NC = 2
NS = 16
NW = NC * NS
CH = 80
NPAD = 10240
RPT = NPAD // NS
NCH_N = N // CH


def _sc_mesh():
    return plsc.VectorSubcoreMesh(core_axis_name="c", subcore_axis_name="s")


SW = 8


@functools.lru_cache(maxsize=None)
def _stats_kernel(E):
    Et = E // NW
    nch = Et // CH

    assert nch % NBUF == 0 and nch // NBUF >= 2

    @functools.partial(
        pl.kernel,
        out_type=(jax.ShapeDtypeStruct((NC, NPAD, SW), jnp.float32),
                  jax.ShapeDtypeStruct((NC, G, SW), jnp.float32)),
        mesh=_sc_mesh(),
        compiler_params=pltpu.CompilerParams(use_tc_tiling_on_sc=False),
        scratch_types=[
            pltpu.VMEM_SHARED((NPAD, SW), jnp.float32),
            pltpu.VMEM_SHARED((G, SW), jnp.float32),
            pltpu.VMEM((nch, CH), jnp.int32),
            pltpu.VMEM((CH,), jnp.int32),
            pltpu.VMEM((CH, SW), jnp.float32),
            pltpu.SemaphoreType.DMA((NBUF,)),
        ],
    )
    def stats_kernel(dst_hbm, seg_hbm, zeros_hbm, ones_hbm, deg_out,
                     cnt_out, deg_s, cnt_s, idx2, idx, ones_b, ssem):
        c = lax.axis_index("c")
        s = lax.axis_index("s")
        w = s * NC + c

        pltpu.sync_copy(ones_hbm, ones_b)
        pltpu.sync_copy(zeros_hbm, deg_s.at[pl.ds(s * RPT, RPT)])

        @pl.when(s == 0)
        def _():
            pltpu.sync_copy(zeros_hbm.at[pl.ds(0, G)], cnt_s)

        rowbase = (c * (E // NC) + s * Et) // CH
        pltpu.sync_copy(dst_hbm.at[pl.ds(rowbase, nch)], idx2)
        plsc.subcore_barrier()

        def start_scatter(j, u):
            pltpu.async_copy(ones_b, deg_s.at[idx2.at[j]], ssem.at[u],
                             add=True)

        def wait_scatter(j, u):
            pltpu.make_async_copy(ones_b, deg_s.at[idx2.at[j]],
                                  ssem.at[u]).wait()

        for u in range(NBUF):
            start_scatter(u, u)

        def steady(t, _):
            for u in range(NBUF):
                j = t * NBUF + u
                wait_scatter(j - NBUF, u)
                start_scatter(j, u)
            return 0

        lax.fori_loop(1, nch // NBUF, steady, 0)
        for u in range(NBUF):
            wait_scatter(nch - NBUF + u, u)

        def cnt_body(j, _):
            base = (w + j * NW) * CH
            pltpu.sync_copy(seg_hbm.at[pl.ds(base, CH)], idx)
            pltpu.sync_copy(ones_b, cnt_s.at[idx], add=True)
            return 0

        lax.fori_loop(0, (NCH_N - w + NW - 1) // NW, cnt_body, 0)
        plsc.subcore_barrier()
        pltpu.sync_copy(deg_s.at[pl.ds(s * RPT, RPT)],
                        deg_out.at[c, pl.ds(s * RPT, RPT)])

        @pl.when(s == 0)
        def _():
            pltpu.sync_copy(cnt_s, cnt_out.at[c])

    return stats_kernel


NBUF = 5


@functools.lru_cache(maxsize=None)
def _agg_kernel(E):
    Et = E // NW
    nch = Et // CH
    assert nch % NBUF == 0 and nch // NBUF >= 2

    @functools.partial(
        pl.kernel,
        out_type=jax.ShapeDtypeStruct((NC, NPAD, H), jnp.float32),
        mesh=_sc_mesh(),
        compiler_params=pltpu.CompilerParams(use_tc_tiling_on_sc=False),
        scratch_types=[
            pltpu.VMEM_SHARED((NPAD, H), jnp.float32),
            pltpu.VMEM((nch, CH), jnp.int32),
            pltpu.VMEM((nch, CH), jnp.int32),
            pltpu.VMEM((NBUF, CH, H), jnp.float32),
            pltpu.VMEM((RPT, H), jnp.float32),
            pltpu.SemaphoreType.DMA((NBUF,)),
            pltpu.SemaphoreType.DMA((NBUF,)),
        ],
    )
    def agg_kernel(y_hbm, src_hbm, dst_hbm, out_hbm, acc_s,
                   idx_s, idx_d, rows, zbuf, gsem, ssem):
        c = lax.axis_index("c")
        s = lax.axis_index("s")

        def fill_z(i, _):
            for k in range(H // 16):
                zbuf[i, pl.ds(k * 16, 16)] = jnp.zeros((16,), jnp.float32)
            return 0

        lax.fori_loop(0, RPT, fill_z, 0)
        pltpu.sync_copy(zbuf, acc_s.at[pl.ds(s * RPT, RPT)])

        rowbase = (c * (E // NC) + s * Et) // CH
        pltpu.sync_copy(src_hbm.at[pl.ds(rowbase, nch)], idx_s)
        pltpu.sync_copy(dst_hbm.at[pl.ds(rowbase, nch)], idx_d)
        plsc.subcore_barrier()

        def start_gather(j, u):
            pltpu.async_copy(y_hbm.at[idx_s.at[j]], rows.at[u], gsem.at[u])

        def wait_gather(j, u):
            pltpu.make_async_copy(y_hbm.at[idx_s.at[j]], rows.at[u],
                                  gsem.at[u]).wait()

        def start_scatter(j, u):
            pltpu.async_copy(rows.at[u], acc_s.at[idx_d.at[j]], ssem.at[u],
                             add=True)

        def wait_scatter(j, u):
            pltpu.make_async_copy(rows.at[u], acc_s.at[idx_d.at[j]],
                                  ssem.at[u]).wait()

        LA = 4
        for u in range(LA):
            start_gather(u, u)
        for u in range(NBUF):
            j = u
            wait_gather(j, u)
            start_scatter(j, u)
            if j + LA < nch:
                u2 = (u + LA) % NBUF
                if j + LA >= NBUF:
                    wait_scatter(j + LA - NBUF, u2)
                start_gather(j + LA, u2)

        def steady(t, _):
            for u in range(NBUF):
                j = t * NBUF + u
                wait_gather(j, u)
                start_scatter(j, u)
                u2 = (u + LA) % NBUF
                wait_scatter(j + LA - NBUF, u2)
                start_gather(j + LA, u2)
            return 0

        lax.fori_loop(1, nch // NBUF - 1, steady, 0)

        for u in range(NBUF):
            j = nch - NBUF + u
            wait_gather(j, u)
            start_scatter(j, u)
            if j + LA < nch:
                u2 = (u + LA) % NBUF
                wait_scatter(j + LA - NBUF, u2)
                start_gather(j + LA, u2)
        for u in range(NBUF):
            wait_scatter(nch - NBUF + u, u)

        plsc.subcore_barrier()
        pltpu.sync_copy(acc_s.at[pl.ds(s * RPT, RPT)],
                        out_hbm.at[c, pl.ds(s * RPT, RPT)])

    return agg_kernel


@functools.lru_cache(maxsize=None)
def _agg_pool_kernel(E):
    Et = E // NW
    nch = Et // CH
    assert nch % NBUF == 0 and nch // NBUF >= 2

    @functools.partial(
        pl.kernel,
        out_type=jax.ShapeDtypeStruct((NC, G, H), jnp.float32),
        mesh=_sc_mesh(),
        compiler_params=pltpu.CompilerParams(use_tc_tiling_on_sc=False),
        scratch_types=[
            pltpu.VMEM_SHARED((NPAD, H), jnp.float32),
            pltpu.VMEM_SHARED((G, H), jnp.float32),
            pltpu.VMEM((nch, CH), jnp.int32),
            pltpu.VMEM((nch, CH), jnp.int32),
            pltpu.VMEM((NBUF, CH, H), jnp.float32),
            pltpu.VMEM((2, CH, 16), jnp.float32),
            pltpu.VMEM((2, CH), jnp.int32),
            pltpu.VMEM((RPT, H), jnp.float32),
            pltpu.SemaphoreType.DMA((NBUF,)),
            pltpu.SemaphoreType.DMA((NBUF,)),
        ],
    )
    def agg_pool_kernel(y_hbm, src_hbm, dst_hbm, dv_hbm, seg_hbm,
                        out_hbm, acc_s, pool_s, idx_s, idx_d,
                        rows, dbuf, sidx, zbuf, gsem, ssem):
        c = lax.axis_index("c")
        s = lax.axis_index("s")

        def fill_z(i, _):
            for k in range(H // 16):
                zbuf[i, pl.ds(k * 16, 16)] = jnp.zeros((16,), jnp.float32)
            return 0

        lax.fori_loop(0, RPT, fill_z, 0)
        pltpu.sync_copy(zbuf, acc_s.at[pl.ds(s * RPT, RPT)])

        @pl.when(s == 0)
        def _():
            pltpu.sync_copy(zbuf.at[pl.ds(0, G)], pool_s)

        rowbase = (c * (E // NC) + s * Et) // CH
        pltpu.sync_copy(src_hbm.at[pl.ds(rowbase, nch)], idx_s)
        pltpu.sync_copy(dst_hbm.at[pl.ds(rowbase, nch)], idx_d)
        plsc.subcore_barrier()

        def start_gather(j, u):
            pltpu.async_copy(y_hbm.at[idx_s.at[j]], rows.at[u], gsem.at[u])

        def wait_gather(j, u):
            pltpu.make_async_copy(y_hbm.at[idx_s.at[j]], rows.at[u],
                                  gsem.at[u]).wait()

        def start_scatter(j, u):
            pltpu.async_copy(rows.at[u], acc_s.at[idx_d.at[j]], ssem.at[u],
                             add=True)

        def wait_scatter(j, u):
            pltpu.make_async_copy(rows.at[u], acc_s.at[idx_d.at[j]],
                                  ssem.at[u]).wait()

        LA = 4
        for u in range(LA):
            start_gather(u, u)
        for u in range(NBUF):
            j = u
            wait_gather(j, u)
            start_scatter(j, u)
            if j + LA < nch:
                u2 = (u + LA) % NBUF
                if j + LA >= NBUF:
                    wait_scatter(j + LA - NBUF, u2)
                start_gather(j + LA, u2)

        def steady(t, _):
            for u in range(NBUF):
                j = t * NBUF + u
                wait_gather(j, u)
                start_scatter(j, u)
                u2 = (u + LA) % NBUF
                wait_scatter(j + LA - NBUF, u2)
                start_gather(j + LA, u2)
            return 0

        lax.fori_loop(1, nch // NBUF - 1, steady, 0)

        for u in range(NBUF):
            j = nch - NBUF + u
            wait_gather(j, u)
            start_scatter(j, u)
            if j + LA < nch:
                u2 = (u + LA) % NBUF
                wait_scatter(j + LA - NBUF, u2)
                start_gather(j + LA, u2)
        for u in range(NBUF):
            wait_scatter(nch - NBUF + u, u)

        plsc.subcore_barrier()

        nk = (NCH_N - s + NS - 1) // NS
        a = rows.at[0]
        yb = rows.at[1]
        d = dbuf.at[0]
        si = sidx.at[0]

        def pool_body(i, _):
            k = s + i * NS
            base = k * CH
            pltpu.sync_copy(acc_s.at[pl.ds(base, CH)], a)
            pltpu.sync_copy(dv_hbm.at[pl.ds(base, CH)], d)
            pltpu.sync_copy(seg_hbm.at[k], si)

            @pl.when(c == 0)
            def _():
                pltpu.sync_copy(y_hbm.at[pl.ds(base, CH)], yb)

                def ew(r2, _):
                    for rr in range(2):
                        r = r2 * 2 + rr
                        dv = d[r, :]
                        for k4 in range(H // 16):
                            sl = pl.ds(k4 * 16, 16)
                            a[r, sl] = (a[r, sl] + yb[r, sl]) * dv
                    return 0

                lax.fori_loop(0, CH // 2, ew, 0)

            @pl.when(c != 0)
            def _():
                def ew(r2, _):
                    for rr in range(2):
                        r = r2 * 2 + rr
                        dv = d[r, :]
                        for k4 in range(H // 16):
                            sl = pl.ds(k4 * 16, 16)
                            a[r, sl] = a[r, sl] * dv
                    return 0

                lax.fori_loop(0, CH // 2, ew, 0)

            pltpu.sync_copy(a, pool_s.at[si], add=True)
            return 0

        lax.fori_loop(0, nk, pool_body, 0)
        plsc.subcore_barrier()

        @pl.when(s == 0)
        def _():
            pltpu.sync_copy(pool_s, out_hbm.at[c])

    return agg_pool_kernel


def _tc_scale_matmul(x, W1, degp):
    def body(x_ref, w_ref, degp_ref, y_ref, dinv_ref):
        deg = degp_ref[0, 0:N, 0:1] + degp_ref[1, 0:N, 0:1] + 1.0
        dinv = lax.rsqrt(deg)
        y_ref[...] = jnp.dot(x_ref[...], w_ref[...],
                             preferred_element_type=jnp.float32) * dinv
        dinv_ref[...] = jnp.broadcast_to(dinv, (N, 16))

    return pl.pallas_call(
        body,
        out_shape=[jax.ShapeDtypeStruct((N, H), jnp.float32),
                   jax.ShapeDtypeStruct((N, 16), jnp.float32)],
    )(x, W1, degp)


def _tc_mid(p, y1, dinv8, b1, W2):
    def body(p_ref, y_ref, dinv_ref, b_ref, w_ref, y2_ref):
        dinv = dinv_ref[:, 0:1]
        h = dinv * (p_ref[0, 0:N, :] + p_ref[1, 0:N, :] + y_ref[...]) + b_ref[...]
        h = jnp.maximum(h, 0.0)
        y2_ref[...] = jnp.dot(h, w_ref[...],
                              preferred_element_type=jnp.float32) * dinv

    return pl.pallas_call(
        body,
        out_shape=jax.ShapeDtypeStruct((N, H), jnp.float32),
    )(p, y1, dinv8, b1, W2)


def _tc_head(poolp, cntp, b2, Wl, bl):
    def body(pp_ref, cnt_ref, b_ref, wl_ref, bl_ref, o_ref):
        cnt = cnt_ref[0, :, 0:1] + cnt_ref[1, :, 0:1]
        sums = pp_ref[0] + pp_ref[1] + cnt * b_ref[...]
        pooled = sums / jnp.maximum(cnt, 1.0)
        o_ref[...] = jnp.dot(pooled, wl_ref[...],
                             preferred_element_type=jnp.float32) + bl_ref[...]

    return pl.pallas_call(
        body,
        out_shape=jax.ShapeDtypeStruct((G, 1), jnp.float32),
    )(poolp, cntp, b2, Wl, bl)


def kernel(x, edge_index, batch, num_graphs, W1, b1, W2, b2, Wl, bl):
    src = edge_index[0]
    dst = edge_index[1]
    E = src.shape[0]
    seg = jnp.minimum(batch, num_graphs - 1).astype(jnp.int32)

    src2 = src.reshape(E // CH, CH)
    dst2 = dst.reshape(E // CH, CH)
    zeros8 = jnp.zeros((RPT, SW), jnp.float32)
    ones8 = jnp.ones((CH, SW), jnp.float32)
    degp, cntp = _stats_kernel(E)(dst2, seg, zeros8, ones8)
    y1, dinvb = _tc_scale_matmul(x, W1, degp)
    p1 = _agg_kernel(E)(y1, src2, dst2)
    y2 = _tc_mid(p1, y1, dinvb, b1.reshape(1, H), W2)
    seg2 = seg.reshape(NCH_N, CH)
    poolp = _agg_pool_kernel(E)(y2, src2, dst2, dinvb, seg2)
    return _tc_head(poolp, cntp, b2.reshape(1, H), Wl, bl.reshape(1, 1))


# speedup vs baseline: 1.2080x; 1.0323x over previous
import functools

import jax
import jax.numpy as jnp
from jax import lax
from jax.experimental import pallas as pl
from jax.experimental.pallas import tpu as pltpu
from jax.experimental.pallas import tpu_sc as plsc

N = 10000
H = 64
G = 128
NC = 2
NS = 16
NW = NC * NS
CH = 80
NPAD = 10240
RPT = NPAD // NS
NCH_N = N // CH


def _sc_mesh():
    return plsc.VectorSubcoreMesh(core_axis_name="c", subcore_axis_name="s")


SW = 8


@functools.lru_cache(maxsize=None)
def _stats_kernel(E):
    Et = E // NW
    nch = Et // CH

    assert nch % NBUF == 0 and nch // NBUF >= 2

    @functools.partial(
        pl.kernel,
        out_type=(jax.ShapeDtypeStruct((NC, NPAD, SW), jnp.float32),
                  jax.ShapeDtypeStruct((NC, G, SW), jnp.float32)),
        mesh=_sc_mesh(),
        compiler_params=pltpu.CompilerParams(use_tc_tiling_on_sc=False),
        scratch_types=[
            pltpu.VMEM_SHARED((NPAD, SW), jnp.float32),
            pltpu.VMEM_SHARED((G, SW), jnp.float32),
            pltpu.VMEM((nch, CH), jnp.int32),
            pltpu.VMEM((CH,), jnp.int32),
            pltpu.VMEM((CH, SW), jnp.float32),
            pltpu.SemaphoreType.DMA((NBUF,)),
        ],
    )
    def stats_kernel(dst_hbm, seg_hbm, zeros_hbm, ones_hbm, deg_out,
                     cnt_out, deg_s, cnt_s, idx2, idx, ones_b, ssem):
        c = lax.axis_index("c")
        s = lax.axis_index("s")
        w = s * NC + c

        pltpu.sync_copy(ones_hbm, ones_b)
        pltpu.sync_copy(zeros_hbm, deg_s.at[pl.ds(s * RPT, RPT)])

        @pl.when(s == 0)
        def _():
            pltpu.sync_copy(zeros_hbm.at[pl.ds(0, G)], cnt_s)

        rowbase = (c * (E // NC) + s * Et) // CH
        pltpu.sync_copy(dst_hbm.at[pl.ds(rowbase, nch)], idx2)
        plsc.subcore_barrier()

        def start_scatter(j, u):
            pltpu.async_copy(ones_b, deg_s.at[idx2.at[j]], ssem.at[u],
                             add=True)

        def wait_scatter(j, u):
            pltpu.make_async_copy(ones_b, deg_s.at[idx2.at[j]],
                                  ssem.at[u]).wait()

        for u in range(NBUF):
            start_scatter(u, u)

        def steady(t, _):
            for u in range(NBUF):
                j = t * NBUF + u
                wait_scatter(j - NBUF, u)
                start_scatter(j, u)
            return 0

        lax.fori_loop(1, nch // NBUF, steady, 0)
        for u in range(NBUF):
            wait_scatter(nch - NBUF + u, u)

        def cnt_body(j, _):
            base = (w + j * NW) * CH
            pltpu.sync_copy(seg_hbm.at[pl.ds(base, CH)], idx)
            pltpu.sync_copy(ones_b, cnt_s.at[idx], add=True)
            return 0

        lax.fori_loop(0, (NCH_N - w + NW - 1) // NW, cnt_body, 0)
        plsc.subcore_barrier()
        pltpu.sync_copy(deg_s.at[pl.ds(s * RPT, RPT)],
                        deg_out.at[c, pl.ds(s * RPT, RPT)])

        @pl.when(s == 0)
        def _():
            pltpu.sync_copy(cnt_s, cnt_out.at[c])

    return stats_kernel


NBUF = 5


@functools.lru_cache(maxsize=None)
def _agg_kernel(E):
    Et = E // NW
    nch = Et // CH
    assert nch % NBUF == 0 and nch // NBUF >= 2

    @functools.partial(
        pl.kernel,
        out_type=jax.ShapeDtypeStruct((NC, NPAD, H), jnp.float32),
        mesh=_sc_mesh(),
        compiler_params=pltpu.CompilerParams(use_tc_tiling_on_sc=False),
        scratch_types=[
            pltpu.VMEM_SHARED((NPAD, H), jnp.float32),
            pltpu.VMEM((nch, CH), jnp.int32),
            pltpu.VMEM((nch, CH), jnp.int32),
            pltpu.VMEM((NBUF, CH, H), jnp.float32),
            pltpu.VMEM((RPT, H), jnp.float32),
            pltpu.SemaphoreType.DMA((NBUF,)),
            pltpu.SemaphoreType.DMA((NBUF,)),
        ],
    )
    def agg_kernel(y_hbm, src_hbm, dst_hbm, out_hbm, acc_s,
                   idx_s, idx_d, rows, zbuf, gsem, ssem):
        c = lax.axis_index("c")
        s = lax.axis_index("s")

        def fill_z(i, _):
            for k in range(H // 16):
                zbuf[i, pl.ds(k * 16, 16)] = jnp.zeros((16,), jnp.float32)
            return 0

        lax.fori_loop(0, RPT, fill_z, 0)
        pltpu.sync_copy(zbuf, acc_s.at[pl.ds(s * RPT, RPT)])

        rowbase = (c * (E // NC) + s * Et) // CH
        pltpu.sync_copy(src_hbm.at[pl.ds(rowbase, nch)], idx_s)
        pltpu.sync_copy(dst_hbm.at[pl.ds(rowbase, nch)], idx_d)
        plsc.subcore_barrier()

        def start_gather(j, u):
            pltpu.async_copy(y_hbm.at[idx_s.at[j]], rows.at[u], gsem.at[u])

        def wait_gather(j, u):
            pltpu.make_async_copy(y_hbm.at[idx_s.at[j]], rows.at[u],
                                  gsem.at[u]).wait()

        def start_scatter(j, u):
            pltpu.async_copy(rows.at[u], acc_s.at[idx_d.at[j]], ssem.at[u],
                             add=True)

        def wait_scatter(j, u):
            pltpu.make_async_copy(rows.at[u], acc_s.at[idx_d.at[j]],
                                  ssem.at[u]).wait()

        LA = 4
        for u in range(LA):
            start_gather(u, u)
        for u in range(NBUF):
            j = u
            wait_gather(j, u)
            start_scatter(j, u)
            if j + LA < nch:
                u2 = (u + LA) % NBUF
                if j + LA >= NBUF:
                    wait_scatter(j + LA - NBUF, u2)
                start_gather(j + LA, u2)

        def steady(t, _):
            for u in range(NBUF):
                j = t * NBUF + u
                wait_gather(j, u)
                start_scatter(j, u)
                u2 = (u + LA) % NBUF
                wait_scatter(j + LA - NBUF, u2)
                start_gather(j + LA, u2)
            return 0

        lax.fori_loop(1, nch // NBUF - 1, steady, 0)

        for u in range(NBUF):
            j = nch - NBUF + u
            wait_gather(j, u)
            start_scatter(j, u)
            if j + LA < nch:
                u2 = (u + LA) % NBUF
                wait_scatter(j + LA - NBUF, u2)
                start_gather(j + LA, u2)
        for u in range(NBUF):
            wait_scatter(nch - NBUF + u, u)

        plsc.subcore_barrier()
        pltpu.sync_copy(acc_s.at[pl.ds(s * RPT, RPT)],
                        out_hbm.at[c, pl.ds(s * RPT, RPT)])

    return agg_kernel


@functools.lru_cache(maxsize=None)
def _agg_pool_kernel(E):
    Et = E // NW
    nch = Et // CH
    assert nch % NBUF == 0 and nch // NBUF >= 2

    @functools.partial(
        pl.kernel,
        out_type=jax.ShapeDtypeStruct((NC, G, H), jnp.float32),
        mesh=_sc_mesh(),
        compiler_params=pltpu.CompilerParams(use_tc_tiling_on_sc=False),
        scratch_types=[
            pltpu.VMEM_SHARED((NPAD, H), jnp.float32),
            pltpu.VMEM_SHARED((G, H), jnp.float32),
            pltpu.VMEM((nch, CH), jnp.int32),
            pltpu.VMEM((nch, CH), jnp.int32),
            pltpu.VMEM((NBUF, CH, H), jnp.float32),
            pltpu.VMEM((2 * CH, H), jnp.float32),
            pltpu.VMEM((2 * CH, H), jnp.float32),
            pltpu.VMEM((2 * CH, 16), jnp.float32),
            pltpu.VMEM((2, CH), jnp.int32),
            pltpu.VMEM((CH, H), jnp.float32),
            pltpu.SemaphoreType.DMA((NBUF,)),
            pltpu.SemaphoreType.DMA((NBUF,)),
        ],
    )
    def agg_pool_kernel(y_hbm, src_hbm, dst_hbm, dv_hbm, seg_hbm,
                        out_hbm, acc_s, pool_s, idx_s, idx_d,
                        rows, pbuf, ybuf, dbf, segb, zbuf, gsem, ssem):
        c = lax.axis_index("c")
        s = lax.axis_index("s")

        def fill_z(i, _):
            for k in range(H // 16):
                zbuf[i, pl.ds(k * 16, 16)] = jnp.zeros((16,), jnp.float32)
            return 0

        lax.fori_loop(0, CH, fill_z, 0)
        for q in range(RPT // CH):
            pltpu.sync_copy(zbuf, acc_s.at[pl.ds(s * RPT + q * CH, CH)])

        @pl.when(s == 0)
        def _():
            pltpu.sync_copy(zbuf, pool_s.at[pl.ds(0, CH)])
            pltpu.sync_copy(zbuf.at[pl.ds(0, G - CH)],
                            pool_s.at[pl.ds(CH, G - CH)])

        rowbase = (c * (E // NC) + s * Et) // CH
        pltpu.sync_copy(src_hbm.at[pl.ds(rowbase, nch)], idx_s)
        pltpu.sync_copy(dst_hbm.at[pl.ds(rowbase, nch)], idx_d)
        plsc.subcore_barrier()

        def start_gather(j, u):
            pltpu.async_copy(y_hbm.at[idx_s.at[j]], rows.at[u], gsem.at[u])

        def wait_gather(j, u):
            pltpu.make_async_copy(y_hbm.at[idx_s.at[j]], rows.at[u],
                                  gsem.at[u]).wait()

        def start_scatter(j, u):
            pltpu.async_copy(rows.at[u], acc_s.at[idx_d.at[j]], ssem.at[u],
                             add=True)

        def wait_scatter(j, u):
            pltpu.make_async_copy(rows.at[u], acc_s.at[idx_d.at[j]],
                                  ssem.at[u]).wait()

        LA = 4
        for u in range(LA):
            start_gather(u, u)
        for u in range(NBUF):
            j = u
            wait_gather(j, u)
            start_scatter(j, u)
            if j + LA < nch:
                u2 = (u + LA) % NBUF
                if j + LA >= NBUF:
                    wait_scatter(j + LA - NBUF, u2)
                start_gather(j + LA, u2)

        def steady(t, _):
            for u in range(NBUF):
                j = t * NBUF + u
                wait_gather(j, u)
                start_scatter(j, u)
                u2 = (u + LA) % NBUF
                wait_scatter(j + LA - NBUF, u2)
                start_gather(j + LA, u2)
            return 0

        lax.fori_loop(1, nch // NBUF - 1, steady, 0)

        for u in range(NBUF):
            j = nch - NBUF + u
            wait_gather(j, u)
            start_scatter(j, u)
            if j + LA < nch:
                u2 = (u + LA) % NBUF
                wait_scatter(j + LA - NBUF, u2)
                start_gather(j + LA, u2)
        for u in range(NBUF):
            wait_scatter(nch - NBUF + u, u)

        plsc.subcore_barrier()

        a0 = s * 7 + jnp.minimum(s, 13)
        npair = 3 + (s < 13).astype(jnp.int32)

        def ew_rows(dst, yv, nrows, with_y):
            def ew(r2, _):
                for rr in range(2):
                    r = r2 * 2 + rr
                    dv = dbf[r, :]
                    for k4 in range(H // 16):
                        sl = pl.ds(k4 * 16, 16)
                        if with_y:
                            dst[r, sl] = (dst[r, sl] + yv[r, sl]) * dv
                        else:
                            dst[r, sl] = dst[r, sl] * dv
                return 0

            lax.fori_loop(0, nrows // 2, ew, 0)

        def pool_pair(t, _):
            k = a0 + 2 * t
            base = k * CH
            pltpu.sync_copy(acc_s.at[pl.ds(base, 2 * CH)], pbuf)
            pltpu.sync_copy(dv_hbm.at[pl.ds(base, 2 * CH)], dbf)
            pltpu.sync_copy(seg_hbm.at[pl.ds(k, 2)], segb)

            @pl.when(c == 0)
            def _():
                pltpu.sync_copy(y_hbm.at[pl.ds(base, 2 * CH)], ybuf)
                ew_rows(pbuf, ybuf, 2 * CH, True)

            @pl.when(c != 0)
            def _():
                ew_rows(pbuf, ybuf, 2 * CH, False)

            pltpu.sync_copy(pbuf.at[pl.ds(0, CH)], pool_s.at[segb.at[0]],
                            add=True)
            pltpu.sync_copy(pbuf.at[pl.ds(CH, CH)], pool_s.at[segb.at[1]],
                            add=True)
            return 0

        lax.fori_loop(0, npair, pool_pair, 0)

        @pl.when(s >= 13)
        def _():
            k = a0 + 6
            base = k * CH
            pltpu.sync_copy(acc_s.at[pl.ds(base, CH)], pbuf.at[pl.ds(0, CH)])
            pltpu.sync_copy(dv_hbm.at[pl.ds(base, CH)], dbf.at[pl.ds(0, CH)])
            pltpu.sync_copy(seg_hbm.at[k], segb.at[0])

            @pl.when(c == 0)
            def _():
                pltpu.sync_copy(y_hbm.at[pl.ds(base, CH)],
                                ybuf.at[pl.ds(0, CH)])
                ew_rows(pbuf, ybuf, CH, True)

            @pl.when(c != 0)
            def _():
                ew_rows(pbuf, ybuf, CH, False)

            pltpu.sync_copy(pbuf.at[pl.ds(0, CH)], pool_s.at[segb.at[0]],
                            add=True)

        plsc.subcore_barrier()

        @pl.when(s == 0)
        def _():
            pltpu.sync_copy(pool_s, out_hbm.at[c])

    return agg_pool_kernel


def _tc_scale_matmul(x, W1, degp):
    def body(x_ref, w_ref, degp_ref, y_ref, dinv_ref):
        deg = degp_ref[0, 0:N, 0:1] + degp_ref[1, 0:N, 0:1] + 1.0
        dinv = lax.rsqrt(deg)
        y_ref[...] = jnp.dot(x_ref[...], w_ref[...],
                             preferred_element_type=jnp.float32) * dinv
        dinv_ref[...] = jnp.broadcast_to(dinv, (N, 16))

    return pl.pallas_call(
        body,
        out_shape=[jax.ShapeDtypeStruct((N, H), jnp.float32),
                   jax.ShapeDtypeStruct((N, 16), jnp.float32)],
    )(x, W1, degp)


def _tc_mid(p, y1, dinv8, b1, W2):
    def body(p_ref, y_ref, dinv_ref, b_ref, w_ref, y2_ref):
        dinv = dinv_ref[:, 0:1]
        h = dinv * (p_ref[0, 0:N, :] + p_ref[1, 0:N, :] + y_ref[...]) + b_ref[...]
        h = jnp.maximum(h, 0.0)
        y2_ref[...] = jnp.dot(h, w_ref[...],
                              preferred_element_type=jnp.float32) * dinv

    return pl.pallas_call(
        body,
        out_shape=jax.ShapeDtypeStruct((N, H), jnp.float32),
    )(p, y1, dinv8, b1, W2)


def _tc_head(poolp, cntp, b2, Wl, bl):
    def body(pp_ref, cnt_ref, b_ref, wl_ref, bl_ref, o_ref):
        cnt = cnt_ref[0, :, 0:1] + cnt_ref[1, :, 0:1]
        sums = pp_ref[0] + pp_ref[1] + cnt * b_ref[...]
        pooled = sums / jnp.maximum(cnt, 1.0)
        o_ref[...] = jnp.dot(pooled, wl_ref[...],
                             preferred_element_type=jnp.float32) + bl_ref[...]

    return pl.pallas_call(
        body,
        out_shape=jax.ShapeDtypeStruct((G, 1), jnp.float32),
    )(poolp, cntp, b2, Wl, bl)


def kernel(x, edge_index, batch, num_graphs, W1, b1, W2, b2, Wl, bl):
    src = edge_index[0]
    dst = edge_index[1]
    E = src.shape[0]
    seg = jnp.minimum(batch, num_graphs - 1).astype(jnp.int32)

    src2 = src.reshape(E // CH, CH)
    dst2 = dst.reshape(E // CH, CH)
    zeros8 = jnp.zeros((RPT, SW), jnp.float32)
    ones8 = jnp.ones((CH, SW), jnp.float32)
    degp, cntp = _stats_kernel(E)(dst2, seg, zeros8, ones8)
    y1, dinvb = _tc_scale_matmul(x, W1, degp)
    p1 = _agg_kernel(E)(y1, src2, dst2)
    y2 = _tc_mid(p1, y1, dinvb, b1.reshape(1, H), W2)
    seg2 = seg.reshape(NCH_N, CH)
    poolp = _agg_pool_kernel(E)(y2, src2, dst2, dinvb, seg2)
    return _tc_head(poolp, cntp, b2.reshape(1, H), Wl, bl.reshape(1, 1))
